# Initial kernel scaffold; baseline (speedup 1.0000x reference)
import functools

import jax
import jax.numpy as jnp
from jax import lax
from jax.experimental import pallas as pl
from jax.experimental.pallas import tpu as pltpu
from jax.experimental.pallas import tpu_sc as plsc

VOCAB = 1000000
EMBED_DIM = 32
BATCH = 4096
HIST = 200

_B = BATCH * HIST
_NC = 2
_NS = 16
_NW = _NC * _NS
_B_PER_W = _B // _NW
_CHUNK = 1600
_N_CHUNKS = _B_PER_W // _CHUNK


def _make_kernel():
    mesh = plsc.VectorSubcoreMesh(core_axis_name="c", subcore_axis_name="s")

    @functools.partial(
        pl.kernel,
        out_type=jax.ShapeDtypeStruct((_B, EMBED_DIM), jnp.float32),
        mesh=mesh,
        scratch_types=[
            pltpu.VMEM((_CHUNK,), jnp.int32),
            pltpu.VMEM((_CHUNK, EMBED_DIM), jnp.float32),
            pltpu.SemaphoreType.DMA,
        ],
    )
    def emb_kernel(table_hbm, idx_hbm, out_hbm, idx_v, rows_v, sem):
        wid = lax.axis_index("s") * _NC + lax.axis_index("c")
        base = wid * _B_PER_W

        @pl.loop(0, _N_CHUNKS)
        def _(i):
            off = base + i * _CHUNK
            pltpu.sync_copy(idx_hbm.at[pl.ds(off, _CHUNK)], idx_v)
            pltpu.async_copy(table_hbm.at[idx_v], rows_v, sem).wait()
            pltpu.sync_copy(rows_v, out_hbm.at[pl.ds(off, _CHUNK)])

    return emb_kernel


_emb_kernel = _make_kernel()


@jax.jit
def kernel(inputs, embedding_matrix):
    idx = inputs.reshape(-1).astype(jnp.int32)
    out = _emb_kernel(embedding_matrix, idx)
    return out.reshape(BATCH, HIST, EMBED_DIM)


# speedup vs baseline: 1.4772x; 1.4772x over previous
import functools

import jax
import jax.numpy as jnp
from jax import lax
from jax.experimental import pallas as pl
from jax.experimental.pallas import tpu as pltpu
from jax.experimental.pallas import tpu_sc as plsc

VOCAB = 1000000
EMBED_DIM = 32
BATCH = 4096
HIST = 200

_B = BATCH * HIST
_NC = 2
_NS = 16
_NW = _NC * _NS
_B_PER_W = _B // _NW
_CHUNK = 1600
_N_CHUNKS = _B_PER_W // _CHUNK


def _make_kernel():
    mesh = plsc.VectorSubcoreMesh(core_axis_name="c", subcore_axis_name="s")

    @functools.partial(
        pl.kernel,
        out_type=jax.ShapeDtypeStruct((_B, EMBED_DIM), jnp.float32),
        mesh=mesh,
        scratch_types=[
            pltpu.VMEM((_CHUNK,), jnp.int32),
            pltpu.VMEM((_CHUNK, EMBED_DIM), jnp.float32),
            pltpu.SemaphoreType.DMA,
        ],
        compiler_params=pltpu.CompilerParams(use_tc_tiling_on_sc=False),
    )
    def emb_kernel(table_hbm, idx_hbm, out_hbm, idx_v, rows_v, sem):
        wid = lax.axis_index("s") * _NC + lax.axis_index("c")
        base = wid * _B_PER_W

        @pl.loop(0, _N_CHUNKS)
        def _(i):
            off = base + i * _CHUNK
            pltpu.sync_copy(idx_hbm.at[pl.ds(off, _CHUNK)], idx_v)
            pltpu.async_copy(table_hbm.at[idx_v], rows_v, sem).wait()
            pltpu.sync_copy(rows_v, out_hbm.at[pl.ds(off, _CHUNK)])

    return emb_kernel


_emb_kernel = _make_kernel()


@jax.jit
def kernel(inputs, embedding_matrix):
    idx = inputs.reshape(-1).astype(jnp.int32)
    out = _emb_kernel(embedding_matrix, idx)
    return out.reshape(BATCH, HIST, EMBED_DIM)


# speedup vs baseline: 1.5001x; 1.0155x over previous
import functools

import jax
import jax.numpy as jnp
from jax import lax
from jax.experimental import pallas as pl
from jax.experimental.pallas import tpu as pltpu
from jax.experimental.pallas import tpu_sc as plsc

VOCAB = 1000000
EMBED_DIM = 32
BATCH = 4096
HIST = 200

_B = BATCH * HIST
_NC = 2
_NS = 16
_NW = _NC * _NS
_B_PER_W = _B // _NW
_CHUNK = 1600
_N_CHUNKS = _B_PER_W // _CHUNK


def _make_kernel():
    mesh = plsc.VectorSubcoreMesh(core_axis_name="c", subcore_axis_name="s")

    @functools.partial(
        pl.kernel,
        out_type=jax.ShapeDtypeStruct((_B, EMBED_DIM), jnp.float32),
        mesh=mesh,
        scratch_types=[
            pltpu.VMEM((2, _CHUNK), jnp.int32),
            pltpu.VMEM((2, _CHUNK, EMBED_DIM), jnp.float32),
            pltpu.SemaphoreType.DMA,
            pltpu.SemaphoreType.DMA,
            pltpu.SemaphoreType.DMA,
            pltpu.SemaphoreType.DMA,
        ],
        compiler_params=pltpu.CompilerParams(use_tc_tiling_on_sc=False),
    )
    def emb_kernel(table_hbm, idx_hbm, out_hbm, idx_v, rows_v,
                   sem_g0, sem_g1, sem_w0, sem_w1):
        wid = lax.axis_index("s") * _NC + lax.axis_index("c")
        base = wid * _B_PER_W
        sem_g = (sem_g0, sem_g1)
        sem_w = (sem_w0, sem_w1)

        gathers = [None] * _N_CHUNKS
        writes = [None] * _N_CHUNKS

        pltpu.sync_copy(idx_hbm.at[pl.ds(base, _CHUNK)], idx_v.at[0])
        gathers[0] = pltpu.async_copy(
            table_hbm.at[idx_v.at[0]], rows_v.at[0], sem_g[0])

        for i in range(_N_CHUNKS):
            b = i % 2
            nb = 1 - b
            if i + 1 < _N_CHUNKS:
                off_n = base + (i + 1) * _CHUNK
                pltpu.sync_copy(idx_hbm.at[pl.ds(off_n, _CHUNK)],
                                idx_v.at[nb])
                if i >= 1:
                    writes[i - 1].wait()
                gathers[i + 1] = pltpu.async_copy(
                    table_hbm.at[idx_v.at[nb]], rows_v.at[nb], sem_g[nb])
            gathers[i].wait()
            off = base + i * _CHUNK
            writes[i] = pltpu.async_copy(
                rows_v.at[b], out_hbm.at[pl.ds(off, _CHUNK)], sem_w[b])

        writes[_N_CHUNKS - 2].wait()
        writes[_N_CHUNKS - 1].wait()

    return emb_kernel


_emb_kernel = _make_kernel()


@jax.jit
def kernel(inputs, embedding_matrix):
    idx = inputs.reshape(-1).astype(jnp.int32)
    out = _emb_kernel(embedding_matrix, idx)
    return out.reshape(BATCH, HIST, EMBED_DIM)
